# baseline (speedup 1.0000x reference)
import functools

import jax
import jax.numpy as jnp
from jax import lax
from jax.experimental import pallas as pl
from jax.experimental.pallas import tpu as pltpu
from jax.experimental.pallas import tpu_sc as plsc

N = 100000
D = 128
B = 100
NW = 32
RPW = N // NW
CH = 125
NCH = RPW // CH
IDS_PAD = 100352


def _nrsqrt(x):
    i = lax.bitcast_convert_type(x, jnp.int32)
    i = jnp.int32(0x5F3759DF) - lax.shift_right_arithmetic(i, 1)
    y = lax.bitcast_convert_type(i, jnp.float32)
    for _ in range(3):
        y = y * (1.5 - 0.5 * x * y * y)
    return y


_BCAST_DNUMS = lax.GatherDimensionNumbers(
    offset_dims=(), collapsed_slice_dims=(0,), start_index_map=(0,))


def _bcast_last(x):
    idx = jnp.full((16, 1), 15, jnp.int32)
    return lax.gather(x, idx, _BCAST_DNUMS, (1,),
                      mode=lax.GatherScatterMode.PROMISE_IN_BOUNDS)


def _sc_body(z_hbm, ids_hbm, outS, outT, outC,
             zbuf0, zbuf1, ubuf0, ubuf1, idxb0, idxb1, idsbuf,
             accS, accT, accC,
             semz0, semz1, semi, semS0, semS1, semT0, semT1, semg):
    cid = lax.axis_index("c")
    sid = lax.axis_index("s")
    wid = cid * 16 + sid
    row0 = wid * RPW
    astart = (row0 // 8) * 8
    off = row0 - astart
    reg = sid * B

    ids_cp = pltpu.async_copy(ids_hbm.at[pl.ds(astart, 3136)], idsbuf, semi)

    zeros16 = jnp.zeros((16,), jnp.float32)

    def zero_body(i, _):
        r = i >> 3
        kk = (i & 7) * 16
        ubuf0[r, pl.ds(kk, 16)] = zeros16
        return 0

    lax.fori_loop(0, B * 8, zero_body, 0)
    zsrc = ubuf0.at[pl.ds(0, B)]
    pltpu.async_copy(zsrc, accS.at[pl.ds(reg, B)], semg).wait()
    pltpu.async_copy(zsrc, accT.at[pl.ds(reg, B)], semg).wait()

    def zero_cnt(i, _):
        accC[pl.ds(i * 16, 16)] = zeros16
        return 0

    lax.fori_loop(0, 8, zero_cnt, 0)

    zbufs = (zbuf0, zbuf1)
    ubufs = (ubuf0, ubuf1)
    idxbs = (idxb0, idxb1)
    semzs = (semz0, semz1)
    semSs = (semS0, semS1)
    semTs = (semT0, semT1)

    lanes = lax.iota(jnp.int32, 16)
    ones16 = jnp.ones((16,), jnp.float32)
    tailmask = lanes < 13

    def start_z(c, p):
        return pltpu.async_copy(
            z_hbm.at[pl.ds(row0 + c * CH, CH)], zbufs[p], semzs[p])

    def wait_z(p):
        pltpu.make_async_copy(
            z_hbm.at[pl.ds(row0, CH)], zbufs[p], semzs[p]).wait()

    def start_S(p):
        return pltpu.async_copy(zbufs[p], accS.at[idxbs[p]], semSs[p],
                                add=True)

    def wait_S(p):
        pltpu.make_async_copy(zbufs[p], accS.at[idxbs[p]], semSs[p]).wait()

    def start_T(p):
        return pltpu.async_copy(ubufs[p], accT.at[idxbs[p]], semTs[p],
                                add=True)

    def wait_T(p):
        pltpu.make_async_copy(ubufs[p], accT.at[idxbs[p]], semTs[p]).wait()

    def build_idx(p, c):
        ib = off + c * CH
        idxb = idxbs[p]
        for j in range(7):
            iv = idsbuf[pl.ds(ib + j * 16, 16)]
            idxb[pl.ds(j * 16, 16)] = iv + reg
            plsc.addupdate_scatter(accC, [iv], ones16)
        iv = idsbuf[pl.ds(ib + 112, 16)]
        plsc.store_scatter(idxb, [jnp.minimum(112 + lanes, 124)], iv + reg,
                           mask=tailmask)
        plsc.addupdate_scatter(accC, [iv], ones16, mask=tailmask)

    def row_load(zbuf, gbase, r):
        v = [zbuf[gbase + r, pl.ds(k * 16, 16)] for k in range(8)]
        ss = v[0] * v[0]
        for k in range(1, 8):
            ss = ss + v[k] * v[k]
        return v, ss

    def row_rinv(ss):
        tot = _bcast_last(plsc.cumsum(ss))
        return jnp.where(tot >= 1e-16, _nrsqrt(tot), jnp.float32(1e8))

    def row_store(ubuf, gbase, r, v, rv):
        for k in range(8):
            ubuf[gbase + r, pl.ds(k * 16, 16)] = v[k] * rv

    def do_rows(zbuf, ubuf, gbase, nrows):
        starts = list(range(0, nrows - 1, 2))
        if nrows % 2:
            starts.append(nrows - 1)
        prev = None
        for a in starts:
            cur = [(a,) + row_load(zbuf, gbase, a)]
            if a + 1 < nrows:
                cur.append((a + 1,) + row_load(zbuf, gbase, a + 1))
            cur = [(r, v, row_rinv(ss)) for (r, v, ss) in cur]
            if prev is not None:
                for (r, v, rv) in prev:
                    row_store(ubuf, gbase, r, v, rv)
            prev = cur
        for (r, v, rv) in prev:
            row_store(ubuf, gbase, r, v, rv)

    def compute_u(p):
        zbuf = zbufs[p]
        ubuf = ubufs[p]

        def grp(g, _):
            do_rows(zbuf, ubuf, g * 16, 16)
            return 0

        lax.fori_loop(0, 7, grp, 0)
        do_rows(zbuf, ubuf, 112, 13)

    def chunk(c, p, guarded=True):
        wait_z(p)
        if guarded:
            pl.when(c >= 2)(lambda: wait_T(p))
        else:
            wait_T(p)
        build_idx(p, c)
        start_S(p)
        compute_u(p)
        start_T(p)
        wait_S(p)
        if guarded:
            def _refill():
                start_z(c + 2, p)
                return None
            pl.when(c <= NCH - 3)(_refill)

    ids_cp.wait()
    start_z(0, 0)
    start_z(1, 1)

    def loop_body(i, _):
        chunk(2 * i, 0)
        chunk(2 * i + 1, 1)
        return 0

    lax.fori_loop(0, 12, loop_body, 0)
    chunk(jnp.int32(NCH - 1), 0, guarded=False)
    wait_T(0)
    wait_T(1)

    pltpu.sync_copy(accS.at[pl.ds(reg, B)], outS.at[wid])
    pltpu.sync_copy(accT.at[pl.ds(reg, B)], outT.at[wid])
    pltpu.sync_copy(accC, outC.at[wid])


_sc_call = functools.partial(
    pl.kernel,
    out_type=(
        jax.ShapeDtypeStruct((NW, B, D), jnp.float32),
        jax.ShapeDtypeStruct((NW, B, D), jnp.float32),
        jax.ShapeDtypeStruct((NW, D), jnp.float32),
    ),
    mesh=plsc.VectorSubcoreMesh(core_axis_name="c", subcore_axis_name="s"),
    compiler_params=pltpu.CompilerParams(
        use_tc_tiling_on_sc=False, needs_layout_passes=False),
    scratch_types=[
        pltpu.VMEM((CH, D), jnp.float32),
        pltpu.VMEM((CH, D), jnp.float32),
        pltpu.VMEM((CH, D), jnp.float32),
        pltpu.VMEM((CH, D), jnp.float32),
        pltpu.VMEM((CH,), jnp.int32),
        pltpu.VMEM((CH,), jnp.int32),
        pltpu.VMEM((3136,), jnp.int32),
        pltpu.VMEM_SHARED((16 * B, D), jnp.float32),
        pltpu.VMEM_SHARED((16 * B, D), jnp.float32),
        pltpu.VMEM((D,), jnp.float32),
        pltpu.SemaphoreType.DMA,
        pltpu.SemaphoreType.DMA,
        pltpu.SemaphoreType.DMA,
        pltpu.SemaphoreType.DMA,
        pltpu.SemaphoreType.DMA,
        pltpu.SemaphoreType.DMA,
        pltpu.SemaphoreType.DMA,
        pltpu.SemaphoreType.DMA,
    ],
)(_sc_body)


def _tc_epilogue(pS_ref, pT_ref, pC_ref, out_ref):
    S = jnp.sum(pS_ref[...], axis=0)
    T = jnp.sum(pT_ref[...], axis=0)
    cnt = jnp.sum(pC_ref[...], axis=0).reshape(D, 1)[:B]
    cntc = jnp.maximum(cnt, 1.0)
    c = S / cntc
    dot = jnp.sum(T * c, axis=1, keepdims=True)
    cn = jnp.maximum(jnp.sqrt(jnp.sum(c * c, axis=1, keepdims=True)), 1e-8)
    cos_mean = dot / (cn * cntc)
    valid = cnt > 1.0
    per = jnp.where(valid, 1.0 - cos_mean, 0.0)
    nv = jnp.sum(valid.astype(jnp.float32))
    out_ref[0, 0] = jnp.sum(per) / jnp.maximum(nv, 1.0)


def kernel(z, poi_to_road_block):
    ids = poi_to_road_block.astype(jnp.int32)
    ids_pad = jnp.concatenate(
        [ids, jnp.zeros((IDS_PAD - N,), jnp.int32)])
    pS, pT, pC = _sc_call(z, ids_pad)
    loss = pl.pallas_call(
        _tc_epilogue,
        out_shape=jax.ShapeDtypeStruct((1, 1), jnp.float32),
        out_specs=pl.BlockSpec(memory_space=pltpu.SMEM),
    )(pS, pT, pC)
    return loss[0, 0]


# speedup vs baseline: 1.6109x; 1.6109x over previous
import functools

import jax
import jax.numpy as jnp
from jax import lax
from jax.experimental import pallas as pl
from jax.experimental.pallas import tpu as pltpu
from jax.experimental.pallas import tpu_sc as plsc

N = 100000
D = 128
B = 100
NW = 32
RPW = N // NW
CH = 125
NCH = RPW // CH
IDS_PAD = 100352
ACC = B * D


def _nrsqrt(x):
    i = lax.bitcast_convert_type(x, jnp.int32)
    i = jnp.int32(0x5F3759DF) - lax.shift_right_arithmetic(i, 1)
    y = lax.bitcast_convert_type(i, jnp.float32)
    for _ in range(3):
        y = y * (1.5 - 0.5 * x * y * y)
    return y


_BCAST_DNUMS = lax.GatherDimensionNumbers(
    offset_dims=(), collapsed_slice_dims=(0,), start_index_map=(0,))


def _bcast_last(x):
    idx = jnp.full((16, 1), 15, jnp.int32)
    return lax.gather(x, idx, _BCAST_DNUMS, (1,),
                      mode=lax.GatherScatterMode.PROMISE_IN_BOUNDS)


def _sc_body(z_hbm, ids_hbm, outS, outT, outC,
             zbuf0, zbuf1, idxb0, idxb1, idsbuf,
             accS, accT, accC,
             semz0, semz1, semi, semS0, semS1, semg):
    cid = lax.axis_index("c")
    sid = lax.axis_index("s")
    wid = cid * 16 + sid
    row0 = wid * RPW
    astart = (row0 // 8) * 8
    off = row0 - astart
    reg = sid * B

    ids_cp = pltpu.async_copy(ids_hbm.at[pl.ds(astart, 3136)], idsbuf, semi)

    zeros16 = jnp.zeros((16,), jnp.float32)

    def zero_body(i, _):
        accT[pl.ds(i * 16, 16)] = zeros16
        zbuf0[i >> 3, pl.ds((i & 7) * 16, 16)] = zeros16
        return 0

    lax.fori_loop(0, ACC // 16, zero_body, 0)
    pltpu.async_copy(zbuf0.at[pl.ds(0, B)], accS.at[pl.ds(reg, B)],
                     semg).wait()

    def zero_cnt(i, _):
        accC[pl.ds(i * 16, 16)] = zeros16
        return 0

    lax.fori_loop(0, 8, zero_cnt, 0)

    zbufs = (zbuf0, zbuf1)
    idxbs = (idxb0, idxb1)
    semzs = (semz0, semz1)
    semSs = (semS0, semS1)

    lanes = lax.iota(jnp.int32, 16)
    ones16 = jnp.ones((16,), jnp.float32)
    tailmask = lanes < 13

    def start_z(c, p):
        return pltpu.async_copy(
            z_hbm.at[pl.ds(row0 + c * CH, CH)], zbufs[p], semzs[p])

    def wait_z(p):
        pltpu.make_async_copy(
            z_hbm.at[pl.ds(row0, CH)], zbufs[p], semzs[p]).wait()

    def start_S(p):
        return pltpu.async_copy(zbufs[p], accS.at[idxbs[p]], semSs[p],
                                add=True)

    def wait_S(p):
        pltpu.make_async_copy(zbufs[p], accS.at[idxbs[p]], semSs[p]).wait()

    def build_idx(p, c):
        ib = off + c * CH
        idxb = idxbs[p]
        for j in range(7):
            iv = idsbuf[pl.ds(ib + j * 16, 16)]
            idxb[pl.ds(j * 16, 16)] = iv + reg
            plsc.addupdate_scatter(accC, [iv], ones16)
        iv = idsbuf[pl.ds(ib + 112, 16)]
        plsc.store_scatter(idxb, [jnp.minimum(112 + lanes, 124)], iv + reg,
                           mask=tailmask)
        plsc.addupdate_scatter(accC, [iv], ones16, mask=tailmask)

    def row_load(zbuf, gbase, r):
        v = [zbuf[gbase + r, pl.ds(k * 16, 16)] for k in range(8)]
        ss = v[0] * v[0]
        for k in range(1, 8):
            ss = ss + v[k] * v[k]
        return v, ss

    def row_rinv(ss):
        tot = _bcast_last(plsc.cumsum(ss))
        return jnp.where(tot >= 1e-16, _nrsqrt(tot), jnp.float32(1e8))

    def row_addT(idv16, r, v, rv):
        sb = idv16[r] * D
        for k in range(8):
            plsc.addupdate(accT.at[pl.ds(sb + k * 16, 16)], v[k] * rv)

    def do_rows(zbuf, idv16, gbase, nrows):
        starts = list(range(0, nrows - 1, 2))
        if nrows % 2:
            starts.append(nrows - 1)
        prev = None
        for a in starts:
            cur = [(a,) + row_load(zbuf, gbase, a)]
            if a + 1 < nrows:
                cur.append((a + 1,) + row_load(zbuf, gbase, a + 1))
            cur = [(r, v, row_rinv(ss)) for (r, v, ss) in cur]
            if prev is not None:
                for (r, v, rv) in prev:
                    row_addT(idv16, r, v, rv)
            prev = cur
        for (r, v, rv) in prev:
            row_addT(idv16, r, v, rv)

    def compute_T(p, c):
        zbuf = zbufs[p]
        ib = off + c * CH

        def grp(g, _):
            idv16 = idsbuf[pl.ds(ib + g * 16, 16)]
            do_rows(zbuf, idv16, g * 16, 16)
            return 0

        lax.fori_loop(0, 7, grp, 0)
        idv16 = idsbuf[pl.ds(ib + 112, 16)]
        do_rows(zbuf, idv16, 112, 13)

    def chunk(c, p, guarded=True):
        wait_z(p)
        build_idx(p, c)
        start_S(p)
        compute_T(p, c)
        wait_S(p)
        if guarded:
            def _refill():
                start_z(c + 2, p)
                return None
            pl.when(c <= NCH - 3)(_refill)

    ids_cp.wait()
    start_z(0, 0)
    start_z(1, 1)

    def loop_body(i, _):
        chunk(2 * i, 0)
        chunk(2 * i + 1, 1)
        return 0

    lax.fori_loop(0, 12, loop_body, 0)
    chunk(jnp.int32(NCH - 1), 0, guarded=False)

    pltpu.sync_copy(accS.at[pl.ds(reg, B)], outS.at[wid])
    pltpu.sync_copy(accT, outT.at[wid])
    pltpu.sync_copy(accC, outC.at[wid])


_sc_call = functools.partial(
    pl.kernel,
    out_type=(
        jax.ShapeDtypeStruct((NW, B, D), jnp.float32),
        jax.ShapeDtypeStruct((NW, ACC), jnp.float32),
        jax.ShapeDtypeStruct((NW, D), jnp.float32),
    ),
    mesh=plsc.VectorSubcoreMesh(core_axis_name="c", subcore_axis_name="s"),
    compiler_params=pltpu.CompilerParams(
        use_tc_tiling_on_sc=False, needs_layout_passes=False),
    scratch_types=[
        pltpu.VMEM((CH, D), jnp.float32),
        pltpu.VMEM((CH, D), jnp.float32),
        pltpu.VMEM((CH,), jnp.int32),
        pltpu.VMEM((CH,), jnp.int32),
        pltpu.VMEM((3136,), jnp.int32),
        pltpu.VMEM_SHARED((16 * B, D), jnp.float32),
        pltpu.VMEM((ACC,), jnp.float32),
        pltpu.VMEM((D,), jnp.float32),
        pltpu.SemaphoreType.DMA,
        pltpu.SemaphoreType.DMA,
        pltpu.SemaphoreType.DMA,
        pltpu.SemaphoreType.DMA,
        pltpu.SemaphoreType.DMA,
        pltpu.SemaphoreType.DMA,
    ],
)(_sc_body)


def _tc_epilogue(pS_ref, pT_ref, pC_ref, out_ref):
    S = jnp.sum(pS_ref[...], axis=0)
    T = jnp.sum(pT_ref[...], axis=0)
    cnt = jnp.sum(pC_ref[...], axis=0).reshape(D, 1)[:B]
    cntc = jnp.maximum(cnt, 1.0)
    c = S / cntc
    dot = jnp.sum(T * c, axis=1, keepdims=True)
    cn = jnp.maximum(jnp.sqrt(jnp.sum(c * c, axis=1, keepdims=True)), 1e-8)
    cos_mean = dot / (cn * cntc)
    valid = cnt > 1.0
    per = jnp.where(valid, 1.0 - cos_mean, 0.0)
    nv = jnp.sum(valid.astype(jnp.float32))
    out_ref[0, 0] = jnp.sum(per) / jnp.maximum(nv, 1.0)


def kernel(z, poi_to_road_block):
    ids = poi_to_road_block.astype(jnp.int32)
    ids_pad = jnp.concatenate(
        [ids, jnp.zeros((IDS_PAD - N,), jnp.int32)])
    pS, pT, pC = _sc_call(z, ids_pad)
    pT3 = pT.reshape(NW, B, D)
    loss = pl.pallas_call(
        _tc_epilogue,
        out_shape=jax.ShapeDtypeStruct((1, 1), jnp.float32),
        out_specs=pl.BlockSpec(memory_space=pltpu.SMEM),
    )(pS, pT3, pC)
    return loss[0, 0]


# speedup vs baseline: 1.9036x; 1.1817x over previous
import functools

import jax
import jax.numpy as jnp
from jax import lax
from jax.experimental import pallas as pl
from jax.experimental.pallas import tpu as pltpu
from jax.experimental.pallas import tpu_sc as plsc

N = 100000
D = 128
B = 100
NW = 32
SCN = 52000
RPW = SCN // NW
CH = 125
NCH = RPW // CH
IDSF = 1640
ACC = B * D
TCB = 800
TGRID = (N - SCN) // TCB
TOFF = SCN // TCB


def _nrsqrt(x):
    i = lax.bitcast_convert_type(x, jnp.int32)
    i = jnp.int32(0x5F3759DF) - lax.shift_right_arithmetic(i, 1)
    y = lax.bitcast_convert_type(i, jnp.float32)
    for _ in range(3):
        y = y * (1.5 - 0.5 * x * y * y)
    return y


_BCAST_DNUMS = lax.GatherDimensionNumbers(
    offset_dims=(), collapsed_slice_dims=(0,), start_index_map=(0,))


def _bcast_last(x):
    idx = jnp.full((16, 1), 15, jnp.int32)
    return lax.gather(x, idx, _BCAST_DNUMS, (1,),
                      mode=lax.GatherScatterMode.PROMISE_IN_BOUNDS)


def _sc_body(z_hbm, ids_hbm, outS, outT, outC,
             zbuf0, zbuf1, idxb0, idxb1, idsbuf,
             accS, accT, accC,
             semz0, semz1, semi, semS0, semS1, semg):
    cid = lax.axis_index("c")
    sid = lax.axis_index("s")
    wid = cid * 16 + sid
    row0 = wid * RPW
    astart = (row0 // 8) * 8
    off = row0 - astart
    reg = sid * B

    ids_cp = pltpu.async_copy(ids_hbm.at[pl.ds(astart, IDSF)], idsbuf, semi)

    zeros16 = jnp.zeros((16,), jnp.float32)

    def zero_body(i, _):
        accT[pl.ds(i * 16, 16)] = zeros16
        zbuf0[i >> 3, pl.ds((i & 7) * 16, 16)] = zeros16
        return 0

    lax.fori_loop(0, ACC // 16, zero_body, 0)
    pltpu.async_copy(zbuf0.at[pl.ds(0, B)], accS.at[pl.ds(reg, B)],
                     semg).wait()

    def zero_cnt(i, _):
        accC[pl.ds(i * 16, 16)] = zeros16
        return 0

    lax.fori_loop(0, 8, zero_cnt, 0)

    zbufs = (zbuf0, zbuf1)
    idxbs = (idxb0, idxb1)
    semzs = (semz0, semz1)
    semSs = (semS0, semS1)

    lanes = lax.iota(jnp.int32, 16)
    ones16 = jnp.ones((16,), jnp.float32)
    tailmask = lanes < 13

    def start_z(c, p):
        return pltpu.async_copy(
            z_hbm.at[pl.ds(row0 + c * CH, CH)], zbufs[p], semzs[p])

    def wait_z(p):
        pltpu.make_async_copy(
            z_hbm.at[pl.ds(row0, CH)], zbufs[p], semzs[p]).wait()

    def start_S(p):
        return pltpu.async_copy(zbufs[p], accS.at[idxbs[p]], semSs[p],
                                add=True)

    def wait_S(p):
        pltpu.make_async_copy(zbufs[p], accS.at[idxbs[p]], semSs[p]).wait()

    def build_idx(p, c):
        ib = off + c * CH
        idxb = idxbs[p]
        for j in range(7):
            iv = idsbuf[pl.ds(ib + j * 16, 16)]
            idxb[pl.ds(j * 16, 16)] = iv + reg
            plsc.addupdate_scatter(accC, [iv], ones16)
        iv = idsbuf[pl.ds(ib + 112, 16)]
        plsc.store_scatter(idxb, [jnp.minimum(112 + lanes, 124)], iv + reg,
                           mask=tailmask)
        plsc.addupdate_scatter(accC, [iv], ones16, mask=tailmask)

    def row_load(zbuf, gbase, r):
        v = [zbuf[gbase + r, pl.ds(k * 16, 16)] for k in range(8)]
        ss = v[0] * v[0]
        for k in range(1, 8):
            ss = ss + v[k] * v[k]
        return v, ss

    def row_rinv(ss):
        tot = _bcast_last(plsc.cumsum(ss))
        return jnp.where(tot >= 1e-16, _nrsqrt(tot), jnp.float32(1e8))

    def row_addT(idv16, r, v, rv):
        sb = idv16[r] * D
        for k in range(8):
            plsc.addupdate(accT.at[pl.ds(sb + k * 16, 16)], v[k] * rv)

    def do_rows(zbuf, idv16, gbase, nrows):
        starts = list(range(0, nrows - 1, 2))
        if nrows % 2:
            starts.append(nrows - 1)
        prev = None
        for a in starts:
            cur = [(a,) + row_load(zbuf, gbase, a)]
            if a + 1 < nrows:
                cur.append((a + 1,) + row_load(zbuf, gbase, a + 1))
            cur = [(r, v, row_rinv(ss)) for (r, v, ss) in cur]
            if prev is not None:
                for (r, v, rv) in prev:
                    row_addT(idv16, r, v, rv)
            prev = cur
        for (r, v, rv) in prev:
            row_addT(idv16, r, v, rv)

    def compute_T(p, c):
        zbuf = zbufs[p]
        ib = off + c * CH

        def grp(g, _):
            idv16 = idsbuf[pl.ds(ib + g * 16, 16)]
            do_rows(zbuf, idv16, g * 16, 16)
            return 0

        lax.fori_loop(0, 7, grp, 0)
        idv16 = idsbuf[pl.ds(ib + 112, 16)]
        do_rows(zbuf, idv16, 112, 13)

    def chunk(c, p, guarded=True):
        wait_z(p)
        build_idx(p, c)
        start_S(p)
        compute_T(p, c)
        wait_S(p)
        if guarded:
            def _refill():
                start_z(c + 2, p)
                return None
            pl.when(c <= NCH - 3)(_refill)

    ids_cp.wait()
    start_z(0, 0)
    start_z(1, 1)

    def loop_body(i, _):
        chunk(2 * i, 0)
        chunk(2 * i + 1, 1)
        return 0

    lax.fori_loop(0, (NCH - 1) // 2, loop_body, 0)
    chunk(jnp.int32(NCH - 1), 0, guarded=False)

    pltpu.sync_copy(accS.at[pl.ds(reg, B)], outS.at[wid])
    pltpu.sync_copy(accT, outT.at[wid])
    pltpu.sync_copy(accC, outC.at[wid])


_sc_call = functools.partial(
    pl.kernel,
    out_type=(
        jax.ShapeDtypeStruct((NW, B, D), jnp.float32),
        jax.ShapeDtypeStruct((NW, ACC), jnp.float32),
        jax.ShapeDtypeStruct((NW, D), jnp.float32),
    ),
    mesh=plsc.VectorSubcoreMesh(core_axis_name="c", subcore_axis_name="s"),
    compiler_params=pltpu.CompilerParams(
        use_tc_tiling_on_sc=False, needs_layout_passes=False),
    scratch_types=[
        pltpu.VMEM((CH, D), jnp.float32),
        pltpu.VMEM((CH, D), jnp.float32),
        pltpu.VMEM((CH,), jnp.int32),
        pltpu.VMEM((CH,), jnp.int32),
        pltpu.VMEM((IDSF,), jnp.int32),
        pltpu.VMEM_SHARED((16 * B, D), jnp.float32),
        pltpu.VMEM((ACC,), jnp.float32),
        pltpu.VMEM((D,), jnp.float32),
        pltpu.SemaphoreType.DMA,
        pltpu.SemaphoreType.DMA,
        pltpu.SemaphoreType.DMA,
        pltpu.SemaphoreType.DMA,
        pltpu.SemaphoreType.DMA,
        pltpu.SemaphoreType.DMA,
    ],
)(_sc_body)


def _tc_partial(z_ref, ids_ref, tS_ref, tT_ref, tC_ref):
    @pl.when(pl.program_id(0) == 0)
    def _init():
        tS_ref[...] = jnp.zeros((B, D), jnp.float32)
        tT_ref[...] = jnp.zeros((B, D), jnp.float32)
        tC_ref[...] = jnp.zeros((B, D), jnp.float32)

    zb = z_ref[...]
    idsb = ids_ref[...].reshape(1, TCB)
    bid = lax.broadcasted_iota(jnp.int32, (B, TCB), 0)
    oh = (bid == idsb).astype(jnp.float32)
    ss = jnp.sum(zb * zb, axis=1, keepdims=True)
    rinv = jnp.where(ss >= 1e-16, lax.rsqrt(ss), jnp.float32(1e8))
    u = zb * rinv
    tS_ref[...] += jnp.dot(oh, zb, preferred_element_type=jnp.float32)
    tT_ref[...] += jnp.dot(oh, u, preferred_element_type=jnp.float32)
    tC_ref[...] += jnp.sum(oh, axis=1, keepdims=True)


_tc_partial_call = functools.partial(
    pl.pallas_call,
    grid=(TGRID,),
    in_specs=[
        pl.BlockSpec((TCB, D), lambda i: (i + TOFF, 0)),
        pl.BlockSpec((1, 1, TCB), lambda i: (i + TOFF, 0, 0)),
    ],
    out_specs=[
        pl.BlockSpec((B, D), lambda i: (0, 0)),
        pl.BlockSpec((B, D), lambda i: (0, 0)),
        pl.BlockSpec((B, D), lambda i: (0, 0)),
    ],
    out_shape=[
        jax.ShapeDtypeStruct((B, D), jnp.float32),
        jax.ShapeDtypeStruct((B, D), jnp.float32),
        jax.ShapeDtypeStruct((B, D), jnp.float32),
    ],
)


def _tc_epilogue(pS_ref, pT_ref, pC_ref, tS_ref, tT_ref, tC_ref, out_ref):
    S = jnp.sum(pS_ref[...], axis=0) + tS_ref[...]
    T = jnp.sum(pT_ref[...], axis=0) + tT_ref[...]
    cnt = (jnp.sum(pC_ref[...], axis=0).reshape(D, 1)[:B]
           + tC_ref[:, :1])
    cntc = jnp.maximum(cnt, 1.0)
    c = S / cntc
    dot = jnp.sum(T * c, axis=1, keepdims=True)
    cn = jnp.maximum(jnp.sqrt(jnp.sum(c * c, axis=1, keepdims=True)), 1e-8)
    cos_mean = dot / (cn * cntc)
    valid = cnt > 1.0
    per = jnp.where(valid, 1.0 - cos_mean, 0.0)
    nv = jnp.sum(valid.astype(jnp.float32))
    out_ref[0, 0] = jnp.sum(per) / jnp.maximum(nv, 1.0)


def kernel(z, poi_to_road_block):
    ids = poi_to_road_block.astype(jnp.int32)
    pS, pT, pC = _sc_call(z, ids)
    tS, tT, tC = _tc_partial_call(_tc_partial)(
        z, ids.reshape(N // TCB, 1, TCB))
    pT3 = pT.reshape(NW, B, D)
    loss = pl.pallas_call(
        _tc_epilogue,
        out_shape=jax.ShapeDtypeStruct((1, 1), jnp.float32),
        out_specs=pl.BlockSpec(memory_space=pltpu.SMEM),
    )(pS, pT3, pC, tS, tT, tC)
    return loss[0, 0]
